# Initial kernel scaffold; baseline (speedup 1.0000x reference)
import jax
import jax.numpy as jnp
from jax.experimental import pallas as pl


def kernel(prompt, embedding):
    raise NotImplementedError("write your pallas kernel here")


# speedup vs baseline: 1.1341x; 1.1341x over previous
import functools

import jax
import jax.numpy as jnp
from jax import lax
from jax.experimental import pallas as pl
from jax.experimental.pallas import tpu as pltpu
from jax.experimental.pallas import tpu_sc as plsc

BATCH = 64
PLEN = 50
ROWS = BATCH * PLEN
D = 24576
NC, NS = 2, 16
NW = NC * NS
R_PER_W = ROWS // NW
NB = 2
NCHUNK = R_PER_W // NB
NPAIR = NCHUNK // 2


def kernel(prompt, embedding):
    idx = prompt.reshape(NW, NCHUNK, NB)
    mesh = plsc.VectorSubcoreMesh(core_axis_name="c", subcore_axis_name="s")

    @functools.partial(
        pl.kernel,
        out_type=jax.ShapeDtypeStruct((ROWS, D), jnp.float32),
        mesh=mesh,
        scratch_types=[
            pltpu.VMEM((NCHUNK, NB), jnp.int32),
            pltpu.VMEM((NB, D), jnp.float32),
            pltpu.VMEM((NB, D), jnp.float32),
            pltpu.SemaphoreType.DMA,
            pltpu.SemaphoreType.DMA,
            pltpu.SemaphoreType.DMA,
        ],
    )
    def run(emb_hbm, idx_hbm, out_hbm, idx_v, buf0, buf1, gsem0, gsem1, wsem):
        wid = lax.axis_index("s") * NC + lax.axis_index("c")
        base = wid * R_PER_W
        pltpu.sync_copy(idx_hbm.at[wid], idx_v)

        def gather(c, buf, sem):
            return pltpu.make_async_copy(emb_hbm.at[idx_v.at[c]], buf, sem)

        def write(c, buf):
            return pltpu.make_async_copy(
                buf, out_hbm.at[pl.ds(base + c * NB, NB)], wsem)

        gather(0, buf0, gsem0).start()
        gather(1, buf1, gsem1).start()

        @pl.loop(0, NPAIR)
        def _(g):
            c0 = g * 2
            c1 = c0 + 1
            gather(c0, buf0, gsem0).wait()
            write(c0, buf0).start()
            write(c0, buf0).wait()

            @pl.when(g + 1 < NPAIR)
            def _():
                gather(c0 + 2, buf0, gsem0).start()

            gather(c1, buf1, gsem1).wait()
            write(c1, buf1).start()
            write(c1, buf1).wait()

            @pl.when(g + 1 < NPAIR)
            def _():
                gather(c1 + 2, buf1, gsem1).start()

    out = run(embedding, idx)
    return out.reshape(BATCH, PLEN, D)


# speedup vs baseline: 1.1374x; 1.0030x over previous
import functools

import jax
import jax.numpy as jnp
from jax import lax
from jax.experimental import pallas as pl
from jax.experimental.pallas import tpu as pltpu
from jax.experimental.pallas import tpu_sc as plsc

BATCH = 64
PLEN = 50
ROWS = BATCH * PLEN
D = 24576
NC, NS = 2, 16
NW = NC * NS
R_PER_W = ROWS // NW
NB = 1
NCHUNK = R_PER_W // NB
NBUF = 4


def kernel(prompt, embedding):
    idx = prompt.reshape(NW, NCHUNK, NB)
    mesh = plsc.VectorSubcoreMesh(core_axis_name="c", subcore_axis_name="s")

    @functools.partial(
        pl.kernel,
        out_type=jax.ShapeDtypeStruct((ROWS, D), jnp.float32),
        mesh=mesh,
        scratch_types=[
            pltpu.VMEM((NCHUNK, NB), jnp.int32),
            pltpu.VMEM((NBUF, NB, D), jnp.float32),
            pltpu.SemaphoreType.DMA,
            pltpu.SemaphoreType.DMA,
        ],
    )
    def run(emb_hbm, idx_hbm, out_hbm, idx_v, bufs, gsem, wsem):
        wid = lax.axis_index("s") * NC + lax.axis_index("c")
        base = wid * R_PER_W
        pltpu.sync_copy(idx_hbm.at[wid], idx_v)

        def gather(c):
            return pltpu.make_async_copy(
                emb_hbm.at[idx_v.at[c]], bufs.at[c % NBUF], gsem)

        def write(c):
            return pltpu.make_async_copy(
                bufs.at[c % NBUF], out_hbm.at[pl.ds(base + c * NB, NB)], wsem)

        gather(0).start()
        gather(1).start()

        @pl.loop(0, NCHUNK)
        def _(c):
            gather(c).wait()
            write(c).start()

            @pl.when(c >= 2)
            def _():
                write(c - 2).wait()

            @pl.when(c + 2 < NCHUNK)
            def _():
                gather(c + 2).start()

        write(NCHUNK - 2).wait()
        write(NCHUNK - 1).wait()

    out = run(embedding, idx)
    return out.reshape(BATCH, PLEN, D)


# speedup vs baseline: 1.9435x; 1.7087x over previous
import functools

import jax
import jax.numpy as jnp
from jax import lax
from jax.experimental import pallas as pl
from jax.experimental.pallas import tpu as pltpu
from jax.experimental.pallas import tpu_sc as plsc

BATCH = 64
PLEN = 50
ROWS = BATCH * PLEN
D = 24576
NC, NS = 2, 16
NW = NC * NS
R_PER_W = ROWS // NW
NB = 1
NCHUNK = R_PER_W // NB
NBUF = 4


def kernel(prompt, embedding):
    idx = prompt.reshape(NW, NCHUNK, NB)
    mesh = plsc.VectorSubcoreMesh(core_axis_name="c", subcore_axis_name="s")

    @functools.partial(
        pl.kernel,
        out_type=jax.ShapeDtypeStruct((BATCH, PLEN, D), jnp.float32),
        mesh=mesh,
        scratch_types=[
            pltpu.VMEM((NCHUNK, NB), jnp.int32),
            pltpu.VMEM((NBUF, NB, D), jnp.float32),
            pltpu.SemaphoreType.DMA,
            pltpu.SemaphoreType.DMA,
        ],
    )
    def run(emb_hbm, idx_hbm, out_hbm, idx_v, bufs, gsem, wsem):
        wid = lax.axis_index("s") * NC + lax.axis_index("c")
        pltpu.sync_copy(idx_hbm.at[wid], idx_v)

        def gather(c):
            return pltpu.make_async_copy(
                emb_hbm.at[idx_v.at[c]], bufs.at[c % NBUF], gsem)

        def write(c):
            b = 2 * wid + c // PLEN
            p = c - (c // PLEN) * PLEN
            return pltpu.make_async_copy(
                bufs.at[c % NBUF], out_hbm.at[b, pl.ds(p, NB)], wsem)

        gather(0).start()
        gather(1).start()

        @pl.loop(0, NCHUNK)
        def _(c):
            gather(c).wait()
            write(c).start()

            @pl.when(c >= 2)
            def _():
                write(c - 2).wait()

            @pl.when(c + 2 < NCHUNK)
            def _():
                gather(c + 2).start()

        write(NCHUNK - 2).wait()
        write(NCHUNK - 1).wait()

    return run(embedding, idx)


# speedup vs baseline: 1.9501x; 1.0034x over previous
import functools

import jax
import jax.numpy as jnp
from jax import lax
from jax.experimental import pallas as pl
from jax.experimental.pallas import tpu as pltpu
from jax.experimental.pallas import tpu_sc as plsc

BATCH = 64
PLEN = 50
ROWS = BATCH * PLEN
D = 24576
NC, NS = 2, 16
NW = NC * NS
R_PER_W = ROWS // NW
GRP = 16
NFULL = R_PER_W // GRP * GRP
IPAD = 128
NBUF = 4


def kernel(prompt, embedding):
    idx = prompt.reshape(NW, R_PER_W)
    idx = jnp.pad(idx, ((0, 0), (0, IPAD - R_PER_W)))
    mesh = plsc.VectorSubcoreMesh(core_axis_name="c", subcore_axis_name="s")

    @functools.partial(
        pl.kernel,
        out_type=jax.ShapeDtypeStruct((BATCH, PLEN, D), jnp.float32),
        mesh=mesh,
        compiler_params=pltpu.CompilerParams(use_tc_tiling_on_sc=True),
        scratch_types=[
            pltpu.VMEM((IPAD,), jnp.int32),
            pltpu.VMEM((D,), jnp.float32),
            pltpu.VMEM((D,), jnp.float32),
            pltpu.VMEM((D,), jnp.float32),
            pltpu.VMEM((D,), jnp.float32),
            pltpu.SemaphoreType.DMA,
            pltpu.SemaphoreType.DMA,
        ],
    )
    def run(emb_hbm, idx_hbm, out_hbm, idx_v, b0, b1, b2, b3, gsem, wsem):
        bufs = (b0, b1, b2, b3)
        wid = lax.axis_index("s") * NC + lax.axis_index("c")
        pltpu.sync_copy(idx_hbm.at[wid], idx_v)

        def gather(row, k):
            return pltpu.make_async_copy(emb_hbm.at[row], bufs[k % NBUF], gsem)

        def write(c, k):
            b = 2 * wid + c // PLEN
            p = c - (c // PLEN) * PLEN
            return pltpu.make_async_copy(bufs[k % NBUF], out_hbm.at[b, p], wsem)

        v0 = idx_v[pl.ds(0, GRP)]
        gather(v0[0], 0).start()
        gather(v0[1], 1).start()

        @pl.loop(0, NFULL, step=GRP)
        def _(c0):
            v = idx_v[pl.ds(c0, GRP)]
            w = idx_v[pl.ds(c0 + GRP, GRP)]
            for k in range(GRP):
                c = c0 + k
                gather(jnp.int32(0), k).wait()
                write(c, k).start()

                @pl.when(c >= 2)
                def _():
                    write(c - 2, k - 2).wait()

                nxt = v[k + 2] if k + 2 < GRP else w[k + 2 - GRP]
                gather(nxt, k + 2).start()

        vt = idx_v[pl.ds(NFULL, GRP)]
        for k in range(R_PER_W - NFULL):
            c = NFULL + k
            gather(jnp.int32(0), k).wait()
            write(c, k).start()
            write(c - 2, k - 2).wait()
            if c + 2 < R_PER_W:
                gather(vt[k + 2], k + 2).start()

        write(R_PER_W - 2, R_PER_W - 2).wait()
        write(R_PER_W - 1, R_PER_W - 1).wait()

    return run(embedding, idx)


# speedup vs baseline: 3.6135x; 1.8530x over previous
import functools

import jax
import jax.numpy as jnp
from jax import lax
from jax.experimental import pallas as pl
from jax.experimental.pallas import tpu as pltpu
from jax.experimental.pallas import tpu_sc as plsc

BATCH = 64
PLEN = 50
ROWS = BATCH * PLEN
D = 24576
NC, NS = 2, 16
NW = NC * NS
R_PER_W = ROWS // NW
GRP = 16
NFULL = R_PER_W // GRP * GRP
IPAD = 128
NBUF = 4


def kernel(prompt, embedding):
    idx = prompt.reshape(NW, R_PER_W)
    idx = jnp.pad(idx, ((0, 0), (0, IPAD - R_PER_W)))
    mesh = plsc.VectorSubcoreMesh(core_axis_name="c", subcore_axis_name="s")

    @functools.partial(
        pl.kernel,
        out_type=jax.ShapeDtypeStruct((PLEN, BATCH, D), jnp.float32),
        mesh=mesh,
        compiler_params=pltpu.CompilerParams(use_tc_tiling_on_sc=True),
        scratch_types=[
            pltpu.VMEM((IPAD,), jnp.int32),
            pltpu.VMEM((D,), jnp.float32),
            pltpu.VMEM((D,), jnp.float32),
            pltpu.VMEM((D,), jnp.float32),
            pltpu.VMEM((D,), jnp.float32),
            pltpu.SemaphoreType.DMA,
            pltpu.SemaphoreType.DMA,
        ],
    )
    def run(emb_hbm, idx_hbm, out_hbm, idx_v, b0, b1, b2, b3, gsem, wsem):
        bufs = (b0, b1, b2, b3)
        wid = lax.axis_index("s") * NC + lax.axis_index("c")
        pltpu.sync_copy(idx_hbm.at[wid], idx_v)

        def gather(row, k):
            return pltpu.make_async_copy(emb_hbm.at[row], bufs[k % NBUF], gsem)

        def write(c, k):
            b = 2 * wid + c // PLEN
            p = c - (c // PLEN) * PLEN
            return pltpu.make_async_copy(bufs[k % NBUF], out_hbm.at[p, b], wsem)

        v0 = idx_v[pl.ds(0, GRP)]
        gather(v0[0], 0).start()
        gather(v0[1], 1).start()

        @pl.loop(0, NFULL, step=GRP)
        def _(c0):
            v = idx_v[pl.ds(c0, GRP)]
            w = idx_v[pl.ds(c0 + GRP, GRP)]
            for k in range(GRP):
                c = c0 + k
                gather(jnp.int32(0), k).wait()
                write(c, k).start()

                @pl.when(c >= 2)
                def _():
                    write(c - 2, k - 2).wait()

                nxt = v[k + 2] if k + 2 < GRP else w[k + 2 - GRP]
                gather(nxt, k + 2).start()

        vt = idx_v[pl.ds(NFULL, GRP)]
        for k in range(R_PER_W - NFULL):
            c = NFULL + k
            gather(jnp.int32(0), k).wait()
            write(c, k).start()
            write(c - 2, k - 2).wait()
            if c + 2 < R_PER_W:
                gather(vt[k + 2], k + 2).start()

        write(R_PER_W - 2, R_PER_W - 2).wait()
        write(R_PER_W - 1, R_PER_W - 1).wait()

    return run(embedding, idx).transpose(1, 0, 2)
